# baseline (speedup 1.0000x reference)
import functools

import jax
import jax.numpy as jnp
from jax import lax
from jax.experimental import pallas as pl
from jax.experimental.pallas import tpu as pltpu
from jax.experimental.pallas import tpu_sc as plsc

N = 10000
E = 320000
NC = 2
NS = 16
NW_E = NC * NS
NW_C = NS

GEO_MP1 = (112, 90, 18, 10112)
GEO_MP2 = (128, 160, 32, 10240)
GEO_MP34 = (128, 80, 40, 10240)


def _mp_sc(table, src3, dst3, w2, Fc, col_split, geo):
    K, B, CH, NPad = geo
    NCHK = B // CH
    RPT = NPad // NS
    mesh = plsc.VectorSubcoreMesh(core_axis_name="c", subcore_axis_name="s")

    @functools.partial(
        pl.kernel,
        out_type=jax.ShapeDtypeStruct((2, NPad, Fc), jnp.float32),
        mesh=mesh,
        compiler_params=pltpu.CompilerParams(use_tc_tiling_on_sc=False),
        scratch_types=[
            pltpu.VMEM((CH, K), jnp.int32),
            pltpu.VMEM((CH, K), jnp.int32),
            pltpu.VMEM((CH * K,), jnp.float32),
            pltpu.VMEM((K, Fc), jnp.float32),
            pltpu.VMEM((K, Fc), jnp.float32),
            pltpu.VMEM_SHARED((NPad, Fc), jnp.float32),
            pltpu.SemaphoreType.DMA,
            pltpu.SemaphoreType.DMA,
        ],
    )
    def mp(table_h, src_h, dst_h, w_h, out_h, src_v, dst_v, w_v, rows0, rows1,
           acc, sem0, sem1):
        c = lax.axis_index("c")
        s = lax.axis_index("s")
        if col_split:
            wid = s
            tb = table_h.at[c]
        else:
            wid = c * NS + s
            tb = table_h.at[0]
        zero16 = jnp.zeros((16,), jnp.float32)

        def zrow(i, carry):
            for j in range(Fc // 16):
                rows0[i, pl.ds(j * 16, 16)] = zero16
            return carry

        lax.fori_loop(0, 80, zrow, 0)
        r0 = s * RPT
        for z in range(RPT // 80):
            pltpu.sync_copy(rows0.at[pl.ds(0, 80)],
                            acc.at[pl.ds(r0 + z * 80, 80)])
        if RPT % 80:
            zt = RPT % 80
            pltpu.sync_copy(rows0.at[pl.ds(0, zt)],
                            acc.at[pl.ds(r0 + (RPT // 80) * 80, zt)])
        plsc.subcore_barrier()

        splat_idx = [jnp.full((16, 1), e, jnp.int32) for e in range(16)]
        gd = lax.GatherDimensionNumbers(
            offset_dims=(), collapsed_slice_dims=(0,), start_index_map=(0,))

        def scale_scatter(rows, bi):
            def group(g, c3):
                wv = w_v[pl.ds(bi * K + g * 16, 16)]
                for e in range(16):
                    spl = lax.gather(
                        wv, splat_idx[e], gd, (1,),
                        mode=lax.GatherScatterMode.PROMISE_IN_BOUNDS)
                    r = g * 16 + e
                    for j in range(Fc // 16):
                        rows[r, pl.ds(j * 16, 16)] = (
                            rows[r, pl.ds(j * 16, 16)] * spl)
                return c3

            lax.fori_loop(0, K // 16, group, 0)
            pltpu.sync_copy(rows, acc.at[dst_v.at[bi]], add=True)

        def start(bi, rows, sem):
            pltpu.async_copy(tb.at[src_v.at[bi]], rows, sem)

        def wait(rows, sem):
            pltpu.make_async_copy(tb.at[src_v.at[0]], rows, sem).wait()

        def chunk(ci, carry):
            pltpu.sync_copy(src_h.at[wid].at[pl.ds(ci * CH, CH)], src_v)
            pltpu.sync_copy(dst_h.at[wid].at[pl.ds(ci * CH, CH)], dst_v)
            pltpu.sync_copy(w_h.at[wid].at[pl.ds(ci * CH * K, CH * K)], w_v)
            start(0, rows0, sem0)

            def pair(pi, c2):
                b0 = 2 * pi
                wait(rows0, sem0)
                start(b0 + 1, rows1, sem1)
                scale_scatter(rows0, b0)
                wait(rows1, sem1)
                start(b0 + 2, rows0, sem0)
                scale_scatter(rows1, b0 + 1)
                return c2

            lax.fori_loop(0, CH // 2 - 1, pair, 0)
            wait(rows0, sem0)
            start(CH - 1, rows1, sem1)
            scale_scatter(rows0, CH - 2)
            wait(rows1, sem1)
            scale_scatter(rows1, CH - 1)
            return carry

        lax.fori_loop(0, NCHK, chunk, 0)
        plsc.subcore_barrier()
        pltpu.sync_copy(acc.at[pl.ds(r0, RPT)],
                        out_h.at[c].at[pl.ds(r0, RPT)])

    return mp(table, src3, dst3, w2)


R_BLK = 1000
GRID = (N // R_BLK,)


def _m1_body(xa, p1, w1a, w2, b2, out):
    u = xa[...] + p1[0] + p1[1]
    h1 = jnp.maximum(jnp.dot(u, w1a[...], preferred_element_type=jnp.float32),
                     0.0)
    g2 = jnp.dot(h1, w2[...], preferred_element_type=jnp.float32) + b2[...]
    out[0] = g2[:, :128]
    out[1] = g2[:, 128:]


def _m1(xa, p1, w1a, w2, b2):
    return pl.pallas_call(
        _m1_body,
        grid=GRID,
        in_specs=[
            pl.BlockSpec((R_BLK, 144), lambda i: (i, 0)),
            pl.BlockSpec((2, R_BLK, 144), lambda i: (0, i, 0)),
            pl.BlockSpec((144, 1024), lambda i: (0, 0)),
            pl.BlockSpec((1024, 256), lambda i: (0, 0)),
            pl.BlockSpec((1, 256), lambda i: (0, 0)),
        ],
        out_specs=pl.BlockSpec((2, R_BLK, 128), lambda i: (0, i, 0)),
        out_shape=jax.ShapeDtypeStruct((2, GEO_MP2[3], 128), jnp.float32),
    )(xa, p1, w1a, w2, b2)


def _m2_body(g2, p2, w3, b3, out):
    h2 = jnp.maximum(g2[...] + p2[...], 0.0)
    ga = jnp.dot(h2[0], w3[...][:128], preferred_element_type=jnp.float32)
    gb = jnp.dot(h2[1], w3[...][128:], preferred_element_type=jnp.float32)
    out[...] = ga + gb + b3[...]


def _m2(g2, p2, w3, b3):
    return pl.pallas_call(
        _m2_body,
        grid=GRID,
        in_specs=[
            pl.BlockSpec((2, R_BLK, 128), lambda i: (0, i, 0)),
            pl.BlockSpec((2, R_BLK, 128), lambda i: (0, i, 0)),
            pl.BlockSpec((256, 64), lambda i: (0, 0)),
            pl.BlockSpec((1, 64), lambda i: (0, 0)),
        ],
        out_specs=pl.BlockSpec((R_BLK, 64), lambda i: (i, 0)),
        out_shape=jax.ShapeDtypeStruct((GEO_MP34[3], 64), jnp.float32),
    )(g2, p2, w3, b3)


def _m3_body(g3, p3, w4p, b4p, out):
    h3 = jnp.maximum(g3[...] + p3[0] + p3[1], 0.0)
    out[...] = jnp.dot(h3, w4p[...],
                       preferred_element_type=jnp.float32) + b4p[...]


def _m3(g3, p3, w4p, b4p):
    return pl.pallas_call(
        _m3_body,
        grid=GRID,
        in_specs=[
            pl.BlockSpec((R_BLK, 64), lambda i: (i, 0)),
            pl.BlockSpec((2, R_BLK, 64), lambda i: (0, i, 0)),
            pl.BlockSpec((64, 16), lambda i: (0, 0)),
            pl.BlockSpec((1, 16), lambda i: (0, 0)),
        ],
        out_specs=pl.BlockSpec((R_BLK, 16), lambda i: (i, 0)),
        out_shape=jax.ShapeDtypeStruct((GEO_MP34[3], 16), jnp.float32),
    )(g3, p3, w4p, b4p)


def _m4_body(g4, p4, out):
    out[...] = (g4[...] + p4[0] + p4[1])[:, :8]


def _m4(g4, p4):
    return pl.pallas_call(
        _m4_body,
        grid=GRID,
        in_specs=[
            pl.BlockSpec((R_BLK, 16), lambda i: (i, 0)),
            pl.BlockSpec((2, R_BLK, 16), lambda i: (0, i, 0)),
        ],
        out_specs=pl.BlockSpec((R_BLK, 8), lambda i: (i, 0)),
        out_shape=jax.ShapeDtypeStruct((N, 8), jnp.float32),
    )(g4, p4)


def _pad_edges(src, dst, w, nw, geo):
    K, B, _, NPad = geo
    per = E // nw
    pad = B * K - per
    src2 = src.reshape(nw, per)
    dst2 = dst.reshape(nw, per)
    w2 = w.reshape(nw, per)
    if pad:
        src2 = jnp.concatenate(
            [src2, jnp.zeros((nw, pad), jnp.int32)], axis=1)
        dst2 = jnp.concatenate(
            [dst2, jnp.full((nw, pad), NPad - 1, jnp.int32)], axis=1)
        w2 = jnp.concatenate([w2, jnp.zeros((nw, pad), jnp.float32)], axis=1)
    return src2.reshape(nw, B, K), dst2.reshape(nw, B, K), w2


def kernel(x, edge_index, edge_weight, W1, b1, W2, b2, W3, b3, W4, b4):
    src = edge_index[0]
    dst = edge_index[1]
    s1, d1, w1 = _pad_edges(src, dst, edge_weight, NW_E, GEO_MP1)
    s2, d2, w2e = _pad_edges(src, dst, edge_weight, NW_C, GEO_MP2)
    s3, d3, w3e = _pad_edges(src, dst, edge_weight, NW_E, GEO_MP34)

    NP1 = GEO_MP1[3]
    x_aug = jnp.concatenate(
        [x, jnp.ones((N, 1), jnp.float32), jnp.zeros((N, 15), jnp.float32),
         ], axis=1)
    x_augp = jnp.pad(x_aug, ((0, NP1 - N), (0, 0)))
    W1a = jnp.concatenate(
        [W1, b1[None, :], jnp.zeros((15, 1024), jnp.float32)], axis=0)
    W4p = jnp.pad(W4, ((0, 0), (0, 8)))
    b4p = jnp.pad(b4, (0, 8))[None, :]

    P1 = _mp_sc(x_augp.reshape(1, NP1, 144), s1, d1, w1,
                Fc=144, col_split=False, geo=GEO_MP1)
    G2 = _m1(x_augp, P1, W1a, W2, b2[None, :])
    P2 = _mp_sc(G2, s2, d2, w2e, Fc=128, col_split=True, geo=GEO_MP2)
    G3 = _m2(G2, P2, W3, b3[None, :])
    P3 = _mp_sc(G3.reshape(1, GEO_MP34[3], 64), s3, d3, w3e,
                Fc=64, col_split=False, geo=GEO_MP34)
    G4 = _m3(G3, P3, W4p, b4p)
    P4 = _mp_sc(G4.reshape(1, GEO_MP34[3], 16), s3, d3, w3e,
                Fc=16, col_split=False, geo=GEO_MP34)
    return _m4(G4, P4)


# speedup vs baseline: 1.6976x; 1.6976x over previous
import functools

import jax
import jax.numpy as jnp
from jax import lax
from jax.experimental import pallas as pl
from jax.experimental.pallas import tpu as pltpu
from jax.experimental.pallas import tpu_sc as plsc

N = 10000
NP = 10240
E = 320000
NC = 2
NS = 16
K = 80
CH = 25
RPT = NP // NS


def _mp_sc(table, src3, dst3, w2, Fc, col_split, three_buf):
    EC = E // NS if col_split else E // (NC * NS)
    B = EC // K
    NCHK = B // CH
    mesh = plsc.VectorSubcoreMesh(core_axis_name="c", subcore_axis_name="s")

    nbuf = 3 if three_buf else 2
    scratch = [
        pltpu.VMEM((CH, K), jnp.int32),
        pltpu.VMEM((CH, K), jnp.int32),
        pltpu.VMEM((CH * K,), jnp.float32),
    ]
    scratch += [pltpu.VMEM((K, Fc), jnp.float32)] * nbuf
    scratch += [pltpu.VMEM_SHARED((NP, Fc), jnp.float32)]
    scratch += [pltpu.SemaphoreType.DMA] * (2 * nbuf if three_buf else 2)

    @functools.partial(
        pl.kernel,
        out_type=jax.ShapeDtypeStruct((2, NP, Fc), jnp.float32),
        mesh=mesh,
        compiler_params=pltpu.CompilerParams(use_tc_tiling_on_sc=False),
        scratch_types=scratch,
    )
    def mp(table_h, src_h, dst_h, w_h, out_h, *scr):
        if three_buf:
            src_v, dst_v, w_v, r0, r1, r2, acc, g0, g1, g2, s0, s1, s2 = scr
        else:
            src_v, dst_v, w_v, r0, r1, acc, g0, g1 = scr
        c = lax.axis_index("c")
        s = lax.axis_index("s")
        if col_split:
            wid = s
            tb = table_h.at[c]
        else:
            wid = c * NS + s
            tb = table_h.at[0]
        zero16 = jnp.zeros((16,), jnp.float32)

        def zrow(i, carry):
            for j in range(Fc // 16):
                r0[i, pl.ds(j * 16, 16)] = zero16
            return carry

        lax.fori_loop(0, K, zrow, 0)
        base = s * RPT
        for z in range(RPT // K):
            pltpu.sync_copy(r0, acc.at[pl.ds(base + z * K, K)])
        plsc.subcore_barrier()

        splat_idx = [jnp.full((16, 1), e, jnp.int32) for e in range(16)]
        gd = lax.GatherDimensionNumbers(
            offset_dims=(), collapsed_slice_dims=(0,), start_index_map=(0,))

        def scale(rows, bi):
            def group(g, c3):
                wv = w_v[pl.ds(bi * K + g * 16, 16)]
                for e in range(16):
                    spl = lax.gather(
                        wv, splat_idx[e], gd, (1,),
                        mode=lax.GatherScatterMode.PROMISE_IN_BOUNDS)
                    r = g * 16 + e
                    for j in range(Fc // 16):
                        rows[r, pl.ds(j * 16, 16)] = (
                            rows[r, pl.ds(j * 16, 16)] * spl)
                return c3

            lax.fori_loop(0, K // 16, group, 0)

        def start(bi, rows, sem):
            pltpu.async_copy(tb.at[src_v.at[bi]], rows, sem)

        def wait_g(rows, sem):
            pltpu.make_async_copy(tb.at[src_v.at[0]], rows, sem).wait()

        def sscat(rows, bi, sem):
            pltpu.async_copy(rows, acc.at[dst_v.at[bi]], sem, add=True)

        def wait_s(rows, sem):
            pltpu.make_async_copy(rows, acc.at[dst_v.at[0]], sem).wait()

        def stage(ci):
            pltpu.sync_copy(src_h.at[wid].at[pl.ds(ci * CH, CH)], src_v)
            pltpu.sync_copy(dst_h.at[wid].at[pl.ds(ci * CH, CH)], dst_v)
            pltpu.sync_copy(w_h.at[wid].at[pl.ds(ci * CH * K, CH * K)], w_v)

        if three_buf:
            def slot(b, cur, gcur, prv, sprv, gprv, nxt, wait_prev):
                wait_g(cur, gcur)
                scale(cur, b)
                if wait_prev:
                    wait_s(prv, sprv)
                if nxt is not None:
                    start(nxt, prv, gprv)
                sc_sem = {id(r0): s0, id(r1): s1, id(r2): s2}[id(cur)]
                sscat(cur, b, sc_sem)

            def chunk(ci, carry):
                stage(ci)
                start(0, r0, g0)
                start(1, r1, g1)
                slot(0, r0, g0, r2, s2, g2, 2, False)
                slot(1, r1, g1, r0, s0, g0, 3, True)

                def triple(ti, c2):
                    b = 3 * ti + 2
                    slot(b, r2, g2, r1, s1, g1, b + 2, True)
                    slot(b + 1, r0, g0, r2, s2, g2, b + 3, True)
                    slot(b + 2, r1, g1, r0, s0, g0, b + 4, True)
                    return c2

                lax.fori_loop(0, (CH - 4) // 3, triple, 0)
                slot(CH - 2, r2, g2, r1, s1, g1, None, True)
                slot(CH - 1, r0, g0, r2, s2, g2, None, True)
                wait_s(r0, s0)
                return carry

            lax.fori_loop(0, NCHK, chunk, 0)
        else:
            def chunk(ci, carry):
                stage(ci)
                start(0, r0, g0)

                def pair(pi, c2):
                    b0 = 2 * pi
                    wait_g(r0, g0)
                    start(b0 + 1, r1, g1)
                    scale(r0, b0)
                    pltpu.sync_copy(r0, acc.at[dst_v.at[b0]], add=True)
                    wait_g(r1, g1)
                    start(b0 + 2, r0, g0)
                    scale(r1, b0 + 1)
                    pltpu.sync_copy(r1, acc.at[dst_v.at[b0 + 1]], add=True)
                    return c2

                lax.fori_loop(0, CH // 2, pair, 0)
                wait_g(r0, g0)
                scale(r0, CH - 1)
                pltpu.sync_copy(r0, acc.at[dst_v.at[CH - 1]], add=True)
                return carry

            lax.fori_loop(0, NCHK, chunk, 0)

        plsc.subcore_barrier()
        pltpu.sync_copy(acc.at[pl.ds(base, RPT)],
                        out_h.at[c].at[pl.ds(base, RPT)])

    return mp(table, src3, dst3, w2)


R_BLK = 1000
GRID = (N // R_BLK,)


def _m1_body(xa, p1, w1a, w2, b2, out):
    u = xa[...] + p1[0] + p1[1]
    h1 = jnp.maximum(jnp.dot(u, w1a[...], preferred_element_type=jnp.float32),
                     0.0)
    g2 = jnp.dot(h1, w2[...], preferred_element_type=jnp.float32) + b2[...]
    out[0] = g2[:, :128]
    out[1] = g2[:, 128:]


def _m1(xa, p1, w1a, w2, b2):
    return pl.pallas_call(
        _m1_body,
        grid=GRID,
        in_specs=[
            pl.BlockSpec((R_BLK, 144), lambda i: (i, 0)),
            pl.BlockSpec((2, R_BLK, 144), lambda i: (0, i, 0)),
            pl.BlockSpec((144, 1024), lambda i: (0, 0)),
            pl.BlockSpec((1024, 256), lambda i: (0, 0)),
            pl.BlockSpec((1, 256), lambda i: (0, 0)),
        ],
        out_specs=pl.BlockSpec((2, R_BLK, 128), lambda i: (0, i, 0)),
        out_shape=jax.ShapeDtypeStruct((2, NP, 128), jnp.float32),
    )(xa, p1, w1a, w2, b2)


def _m2_body(g2, p2, w3, b3, out):
    h2 = jnp.maximum(g2[...] + p2[...], 0.0)
    ga = jnp.dot(h2[0], w3[...][:128], preferred_element_type=jnp.float32)
    gb = jnp.dot(h2[1], w3[...][128:], preferred_element_type=jnp.float32)
    out[...] = ga + gb + b3[...]


def _m2(g2, p2, w3, b3):
    return pl.pallas_call(
        _m2_body,
        grid=GRID,
        in_specs=[
            pl.BlockSpec((2, R_BLK, 128), lambda i: (0, i, 0)),
            pl.BlockSpec((2, R_BLK, 128), lambda i: (0, i, 0)),
            pl.BlockSpec((256, 64), lambda i: (0, 0)),
            pl.BlockSpec((1, 64), lambda i: (0, 0)),
        ],
        out_specs=pl.BlockSpec((R_BLK, 64), lambda i: (i, 0)),
        out_shape=jax.ShapeDtypeStruct((NP, 64), jnp.float32),
    )(g2, p2, w3, b3)


def _m3_body(g3, p3, w4p, b4p, out):
    h3 = jnp.maximum(g3[...] + p3[0] + p3[1], 0.0)
    out[...] = jnp.dot(h3, w4p[...],
                       preferred_element_type=jnp.float32) + b4p[...]


def _m3(g3, p3, w4p, b4p):
    return pl.pallas_call(
        _m3_body,
        grid=GRID,
        in_specs=[
            pl.BlockSpec((R_BLK, 64), lambda i: (i, 0)),
            pl.BlockSpec((2, R_BLK, 64), lambda i: (0, i, 0)),
            pl.BlockSpec((64, 16), lambda i: (0, 0)),
            pl.BlockSpec((1, 16), lambda i: (0, 0)),
        ],
        out_specs=pl.BlockSpec((R_BLK, 16), lambda i: (i, 0)),
        out_shape=jax.ShapeDtypeStruct((NP, 16), jnp.float32),
    )(g3, p3, w4p, b4p)


def _m4_body(g4, p4, out):
    out[...] = (g4[...] + p4[0] + p4[1])[:, :8]


def _m4(g4, p4):
    return pl.pallas_call(
        _m4_body,
        grid=GRID,
        in_specs=[
            pl.BlockSpec((R_BLK, 16), lambda i: (i, 0)),
            pl.BlockSpec((2, R_BLK, 16), lambda i: (0, i, 0)),
        ],
        out_specs=pl.BlockSpec((R_BLK, 8), lambda i: (i, 0)),
        out_shape=jax.ShapeDtypeStruct((N, 8), jnp.float32),
    )(g4, p4)


def kernel(x, edge_index, edge_weight, W1, b1, W2, b2, W3, b3, W4, b4):
    src = edge_index[0]
    dst = edge_index[1]
    src32 = src.reshape(NC * NS, -1, K)
    dst32 = dst.reshape(NC * NS, -1, K)
    w32 = edge_weight.reshape(NC * NS, -1)
    src16 = src.reshape(NS, -1, K)
    dst16 = dst.reshape(NS, -1, K)
    w16 = edge_weight.reshape(NS, -1)

    x_aug = jnp.concatenate(
        [x, jnp.ones((N, 1), jnp.float32), jnp.zeros((N, 15), jnp.float32)],
        axis=1)
    x_augp = jnp.pad(x_aug, ((0, NP - N), (0, 0)))
    W1a = jnp.concatenate(
        [W1, b1[None, :], jnp.zeros((15, 1024), jnp.float32)], axis=0)
    W4p = jnp.pad(W4, ((0, 0), (0, 8)))
    b4p = jnp.pad(b4, (0, 8))[None, :]

    P1 = _mp_sc(x_augp.reshape(1, NP, 144), src32, dst32, w32,
                Fc=144, col_split=False, three_buf=False)
    G2 = _m1(x_augp, P1, W1a, W2, b2[None, :])
    P2 = _mp_sc(G2, src16, dst16, w16, Fc=128, col_split=True,
                three_buf=True)
    G3 = _m2(G2, P2, W3, b3[None, :])
    P3 = _mp_sc(G3.reshape(1, NP, 64), src32, dst32, w32,
                Fc=64, col_split=False, three_buf=True)
    G4 = _m3(G3, P3, W4p, b4p)
    P4 = _mp_sc(G4.reshape(1, NP, 16), src32, dst32, w32,
                Fc=16, col_split=False, three_buf=True)
    return _m4(G4, P4)
